# baseline (speedup 1.0000x reference)
import functools

import jax
import jax.numpy as jnp
from jax import lax
from jax.experimental import pallas as pl
from jax.experimental.pallas import tpu as pltpu
from jax.experimental.pallas import tpu_sc as plsc

_BATCH = 4096
_SEQ = 200
_DIM = 64
_B = _BATCH * _SEQ

_CHUNK = 512
_NBUF = 2
_NSUB = 4
_SUB = _CHUNK // _NSUB


@functools.cache
def _build(nw: int, nc: int):
    b_per_w = _B // nw
    n_chunks = b_per_w // _CHUNK
    n_outer = n_chunks // _NBUF
    mesh = plsc.VectorSubcoreMesh(core_axis_name="c", subcore_axis_name="s")

    scratch = (
        [pltpu.VMEM((_CHUNK,), jnp.int32) for _ in range(_NBUF)]
        + [pltpu.VMEM((_CHUNK, _DIM), jnp.float32) for _ in range(_NBUF)]
        + [pltpu.SemaphoreType.DMA for _ in range(3 * _NBUF)]
    )

    @functools.partial(
        pl.kernel,
        out_type=jax.ShapeDtypeStruct((_B, _DIM), jnp.float32),
        mesh=mesh,
        scratch_types=scratch,
        compiler_params=pltpu.CompilerParams(use_tc_tiling_on_sc=False),
    )
    def gather_kernel(x_hbm, table_hbm, out_hbm, *scr):
        idx_v = scr[:_NBUF]
        rows_v = scr[_NBUF:2 * _NBUF]
        isem = scr[2 * _NBUF:3 * _NBUF]
        gsem = scr[3 * _NBUF:4 * _NBUF]
        ssem = scr[4 * _NBUF:5 * _NBUF]

        wid = lax.axis_index("s") * nc + lax.axis_index("c")
        base = wid * b_per_w

        def idx_copy(c, b):
            return pltpu.make_async_copy(
                x_hbm.at[pl.ds(base + c * _CHUNK, _CHUNK)], idx_v[b], isem[b])

        def gather_copies(b):
            return [
                pltpu.make_async_copy(
                    table_hbm.at[idx_v[b].at[pl.ds(s * _SUB, _SUB)]],
                    rows_v[b].at[pl.ds(s * _SUB, _SUB)], gsem[b])
                for s in range(_NSUB)
            ]

        def store_copy(c, b):
            return pltpu.make_async_copy(
                rows_v[b], out_hbm.at[pl.ds(base + c * _CHUNK, _CHUNK)], ssem[b])

        for b in range(_NBUF):
            idx_copy(b, b).start()

        @pl.loop(0, n_outer)
        def _outer(g):
            for b in range(_NBUF):
                c = g * _NBUF + b
                idx_copy(c, b).wait()

                @pl.when(g >= 1)
                def _():
                    store_copy(c - _NBUF, b).wait()

                for cp in gather_copies(b):
                    cp.start()
                for cp in gather_copies(b):
                    cp.wait()

                @pl.when(g < n_outer - 1)
                def _():
                    idx_copy(c + _NBUF, b).start()

                store_copy(c, b).start()

        for b in range(_NBUF):
            store_copy((n_outer - 1) * _NBUF + b, b).wait()

    return gather_kernel


def kernel(x, input_embedding):
    info = plsc.get_sparse_core_info()
    nw = info.num_cores * info.num_subcores
    flat_idx = x.reshape(_B).astype(jnp.int32)
    out = _build(nw, info.num_cores)(flat_idx, input_embedding)
    return out.reshape(_BATCH, _SEQ, _DIM)


# speedup vs baseline: 1.2195x; 1.2195x over previous
import functools

import jax
import jax.numpy as jnp
from jax import lax
from jax.experimental import pallas as pl
from jax.experimental.pallas import tpu as pltpu
from jax.experimental.pallas import tpu_sc as plsc

_BATCH = 4096
_SEQ = 200
_DIM = 64
_PAD = 128
_B = _BATCH * _SEQ

_CHUNK = 256
_NBUF = 2


@functools.cache
def _build(nw: int, nc: int):
    b_per_w = _B // nw
    n_chunks = b_per_w // _CHUNK
    n_outer = n_chunks // _NBUF
    mesh = plsc.VectorSubcoreMesh(core_axis_name="c", subcore_axis_name="s")

    scratch = (
        [pltpu.VMEM((_CHUNK,), jnp.int32) for _ in range(_NBUF)]
        + [pltpu.VMEM((_CHUNK, _PAD), jnp.float32) for _ in range(_NBUF)]
        + [pltpu.SemaphoreType.DMA for _ in range(3 * _NBUF)]
    )

    @functools.partial(
        pl.kernel,
        out_type=jax.ShapeDtypeStruct((_B, _PAD), jnp.float32),
        mesh=mesh,
        scratch_types=scratch,
        compiler_params=pltpu.CompilerParams(use_tc_tiling_on_sc=False),
    )
    def gather_kernel(x_hbm, table_hbm, out_hbm, *scr):
        idx_v = scr[:_NBUF]
        rows_v = scr[_NBUF:2 * _NBUF]
        isem = scr[2 * _NBUF:3 * _NBUF]
        gsem = scr[3 * _NBUF:4 * _NBUF]
        ssem = scr[4 * _NBUF:5 * _NBUF]

        wid = lax.axis_index("s") * nc + lax.axis_index("c")
        base = wid * b_per_w

        def idx_copy(c, b):
            return pltpu.make_async_copy(
                x_hbm.at[pl.ds(base + c * _CHUNK, _CHUNK)], idx_v[b], isem[b])

        def gather_copy(b):
            return pltpu.make_async_copy(table_hbm.at[idx_v[b]], rows_v[b], gsem[b])

        def store_copy(c, b):
            return pltpu.make_async_copy(
                rows_v[b], out_hbm.at[pl.ds(base + c * _CHUNK, _CHUNK)], ssem[b])

        for b in range(_NBUF):
            idx_copy(b, b).start()

        @pl.loop(0, n_outer)
        def _outer(g):
            for b in range(_NBUF):
                c = g * _NBUF + b
                idx_copy(c, b).wait()

                @pl.when(g >= 1)
                def _():
                    store_copy(c - _NBUF, b).wait()

                gather_copy(b).start()
                gather_copy(b).wait()

                @pl.when(g < n_outer - 1)
                def _():
                    idx_copy(c + _NBUF, b).start()

                store_copy(c, b).start()

        for b in range(_NBUF):
            store_copy((n_outer - 1) * _NBUF + b, b).wait()

    return gather_kernel


def kernel(x, input_embedding):
    info = plsc.get_sparse_core_info()
    nw = info.num_cores * info.num_subcores
    flat_idx = x.reshape(_B).astype(jnp.int32)
    table_pad = jnp.pad(input_embedding, ((0, 0), (0, _PAD - _DIM)))
    out = _build(nw, info.num_cores)(flat_idx, table_pad)
    return out[:, :_DIM].reshape(_BATCH, _SEQ, _DIM)
